# baseline (speedup 1.0000x reference)
import functools

import jax
import jax.numpy as jnp
from jax.experimental import pallas as pl


def _gcn_body(adj_ref, x_ref, w1_ref, b1_ref, w2_ref, b2_ref, out_ref, *, n):
    adj = adj_ref[...]
    ii = jax.lax.broadcasted_iota(jnp.int32, (n, n), 0)
    jj = jax.lax.broadcasted_iota(jnp.int32, (n, n), 1)
    upper = ii < jj
    eye = ii == jj
    a = ((upper[None, :, :] & (adj != 0)) | eye[None, :, :]).astype(jnp.float32)
    deg = jnp.sum(a, axis=1)
    dinv = jax.lax.rsqrt(deg)
    an = a * (dinv[:, :, None] * dinv[:, None, :])

    def conv(xin, w_ref, b_ref):
        xw = jax.lax.dot_general(
            xin, w_ref[...], (((2,), (0,)), ((), ())),
            preferred_element_type=jnp.float32,
        )
        agg = jax.lax.dot_general(
            an, xw, (((1,), (1,)), ((0,), (0,))),
            preferred_element_type=jnp.float32,
        )
        return agg + b_ref[...][None, :, :]

    h = jax.nn.relu(conv(x_ref[...], w1_ref, b1_ref))
    out2 = conv(h, w2_ref, b2_ref)
    out_ref[:, 0, :] = jnp.max(out2, axis=1)


def kernel(adj_mat, v_feature, W1, b1, W2, b2):
    B, N, _ = adj_mat.shape
    d_in, d_hidden = W1.shape
    d_out = W2.shape[1]
    GB = 8
    adj = adj_mat.astype(jnp.int32)
    b1r = b1.reshape(1, d_hidden).astype(jnp.float32)
    b2r = b2.reshape(1, d_out).astype(jnp.float32)
    out = pl.pallas_call(
        functools.partial(_gcn_body, n=N),
        grid=(B // GB,),
        in_specs=[
            pl.BlockSpec((GB, N, N), lambda i: (i, 0, 0)),
            pl.BlockSpec((GB, N, d_in), lambda i: (i, 0, 0)),
            pl.BlockSpec((d_in, d_hidden), lambda i: (0, 0)),
            pl.BlockSpec((1, d_hidden), lambda i: (0, 0)),
            pl.BlockSpec((d_hidden, d_out), lambda i: (0, 0)),
            pl.BlockSpec((1, d_out), lambda i: (0, 0)),
        ],
        out_specs=pl.BlockSpec((GB, 1, d_out), lambda i: (i, 0, 0)),
        out_shape=jax.ShapeDtypeStruct((B, 1, d_out), jnp.float32),
    )(adj, v_feature.astype(jnp.float32), W1, b1r, W2, b2r)
    return out


# speedup vs baseline: 1.0518x; 1.0518x over previous
import functools

import jax
import jax.numpy as jnp
from jax.experimental import pallas as pl


def _gcn_body(adj_ref, x_ref, w1_ref, b1_ref, w2_ref, b2_ref, out_ref, *, n):
    adj = adj_ref[...]
    ii = jax.lax.broadcasted_iota(jnp.int32, (n, n), 0)
    jj = jax.lax.broadcasted_iota(jnp.int32, (n, n), 1)
    upper = ii < jj
    eye = ii == jj
    a = ((upper[None, :, :] & (adj != 0)) | eye[None, :, :]).astype(jnp.bfloat16)
    deg = jnp.sum(a.astype(jnp.float32), axis=1)
    dinv = jax.lax.rsqrt(deg)

    def conv(xin, w_ref, b_ref):
        xw = jax.lax.dot_general(
            xin, w_ref[...].astype(jnp.bfloat16), (((2,), (0,)), ((), ())),
            preferred_element_type=jnp.float32,
        )
        y = (xw * dinv[:, :, None]).astype(jnp.bfloat16)
        agg = jax.lax.dot_general(
            a, y, (((1,), (1,)), ((0,), (0,))),
            preferred_element_type=jnp.float32,
        )
        return agg * dinv[:, :, None] + b_ref[...][None, :, :]

    h = jax.nn.relu(conv(x_ref[...].astype(jnp.bfloat16), w1_ref, b1_ref))
    out2 = conv(h.astype(jnp.bfloat16), w2_ref, b2_ref)
    out_ref[:, 0, :] = jnp.max(out2, axis=1)


def kernel(adj_mat, v_feature, W1, b1, W2, b2):
    B, N, _ = adj_mat.shape
    d_in, d_hidden = W1.shape
    d_out = W2.shape[1]
    GB = 8
    adj = adj_mat.astype(jnp.int32)
    b1r = b1.reshape(1, d_hidden).astype(jnp.float32)
    b2r = b2.reshape(1, d_out).astype(jnp.float32)
    out = pl.pallas_call(
        functools.partial(_gcn_body, n=N),
        grid=(B // GB,),
        in_specs=[
            pl.BlockSpec((GB, N, N), lambda i: (i, 0, 0)),
            pl.BlockSpec((GB, N, d_in), lambda i: (i, 0, 0)),
            pl.BlockSpec((d_in, d_hidden), lambda i: (0, 0)),
            pl.BlockSpec((1, d_hidden), lambda i: (0, 0)),
            pl.BlockSpec((d_hidden, d_out), lambda i: (0, 0)),
            pl.BlockSpec((1, d_out), lambda i: (0, 0)),
        ],
        out_specs=pl.BlockSpec((GB, 1, d_out), lambda i: (i, 0, 0)),
        out_shape=jax.ShapeDtypeStruct((B, 1, d_out), jnp.float32),
    )(adj, v_feature.astype(jnp.float32), W1, b1r, W2, b2r)
    return out


# speedup vs baseline: 1.4960x; 1.4224x over previous
import functools

import jax
import jax.numpy as jnp
from jax.experimental import pallas as pl


def _gcn_body(adj_ref, x_ref, w1_ref, b1_ref, w2_ref, b2_ref, out_ref, *, n):
    adj = adj_ref[...]
    ii = jax.lax.broadcasted_iota(jnp.int32, (n, n), 0)
    jj = jax.lax.broadcasted_iota(jnp.int32, (n, n), 1)
    upper = ii < jj
    eye = ii == jj
    a = ((upper[None, :, :] & (adj != 0)) | eye[None, :, :]).astype(jnp.bfloat16)
    deg = jnp.sum(a.astype(jnp.float32), axis=1)
    dinv = jax.lax.rsqrt(deg)

    def conv(xin, w_ref, b_ref):
        xw = jax.lax.dot_general(
            xin, w_ref[...].astype(jnp.bfloat16), (((2,), (0,)), ((), ())),
            preferred_element_type=jnp.float32,
        )
        y = (xw * dinv[:, :, None]).astype(jnp.bfloat16)
        agg = jax.lax.dot_general(
            a, y, (((1,), (1,)), ((0,), (0,))),
            preferred_element_type=jnp.float32,
        )
        return agg * dinv[:, :, None] + b_ref[...][None, :, :]

    h = jax.nn.relu(conv(x_ref[...].astype(jnp.bfloat16), w1_ref, b1_ref))
    out2 = conv(h.astype(jnp.bfloat16), w2_ref, b2_ref)
    out_ref[:, 0, :] = jnp.max(out2, axis=1)


def kernel(adj_mat, v_feature, W1, b1, W2, b2):
    B, N, _ = adj_mat.shape
    d_in, d_hidden = W1.shape
    d_out = W2.shape[1]
    GB = 16
    adj = adj_mat.astype(jnp.int32)
    b1r = b1.reshape(1, d_hidden).astype(jnp.float32)
    b2r = b2.reshape(1, d_out).astype(jnp.float32)
    out = pl.pallas_call(
        functools.partial(_gcn_body, n=N),
        grid=(B // GB,),
        in_specs=[
            pl.BlockSpec((GB, N, N), lambda i: (i, 0, 0)),
            pl.BlockSpec((GB, N, d_in), lambda i: (i, 0, 0)),
            pl.BlockSpec((d_in, d_hidden), lambda i: (0, 0)),
            pl.BlockSpec((1, d_hidden), lambda i: (0, 0)),
            pl.BlockSpec((d_hidden, d_out), lambda i: (0, 0)),
            pl.BlockSpec((1, d_out), lambda i: (0, 0)),
        ],
        out_specs=pl.BlockSpec((GB, 1, d_out), lambda i: (i, 0, 0)),
        out_shape=jax.ShapeDtypeStruct((B, 1, d_out), jnp.float32),
    )(adj, v_feature.astype(jnp.float32), W1, b1r, W2, b2r)
    return out


# speedup vs baseline: 1.7056x; 1.1401x over previous
import functools

import jax
import jax.numpy as jnp
from jax.experimental import pallas as pl


def _gcn_body(adj_ref, x_ref, w1_ref, b1_ref, w2_ref, b2_ref, out_ref, *, n):
    adj = adj_ref[...]
    ii = jax.lax.broadcasted_iota(jnp.int32, (n, n), 0)
    jj = jax.lax.broadcasted_iota(jnp.int32, (n, n), 1)
    upper = ii < jj
    eye = ii == jj
    a = ((upper[None, :, :] & (adj != 0)) | eye[None, :, :]).astype(jnp.bfloat16)
    deg = jnp.sum(a.astype(jnp.float32), axis=1)
    dinv = jax.lax.rsqrt(deg)

    def conv(xin, w_ref, b_ref):
        xw = jax.lax.dot_general(
            xin, w_ref[...].astype(jnp.bfloat16), (((2,), (0,)), ((), ())),
            preferred_element_type=jnp.float32,
        )
        y = (xw * dinv[:, :, None]).astype(jnp.bfloat16)
        agg = jax.lax.dot_general(
            a, y, (((1,), (1,)), ((0,), (0,))),
            preferred_element_type=jnp.float32,
        )
        return agg * dinv[:, :, None] + b_ref[...][None, :, :]

    h = jax.nn.relu(conv(x_ref[...].astype(jnp.bfloat16), w1_ref, b1_ref))
    out2 = conv(h.astype(jnp.bfloat16), w2_ref, b2_ref)
    out_ref[:, 0, :] = jnp.max(out2, axis=1)


def kernel(adj_mat, v_feature, W1, b1, W2, b2):
    B, N, _ = adj_mat.shape
    d_in, d_hidden = W1.shape
    d_out = W2.shape[1]
    GB = 32
    adj = adj_mat.astype(jnp.int32)
    b1r = b1.reshape(1, d_hidden).astype(jnp.float32)
    b2r = b2.reshape(1, d_out).astype(jnp.float32)
    out = pl.pallas_call(
        functools.partial(_gcn_body, n=N),
        grid=(B // GB,),
        in_specs=[
            pl.BlockSpec((GB, N, N), lambda i: (i, 0, 0)),
            pl.BlockSpec((GB, N, d_in), lambda i: (i, 0, 0)),
            pl.BlockSpec((d_in, d_hidden), lambda i: (0, 0)),
            pl.BlockSpec((1, d_hidden), lambda i: (0, 0)),
            pl.BlockSpec((d_hidden, d_out), lambda i: (0, 0)),
            pl.BlockSpec((1, d_out), lambda i: (0, 0)),
        ],
        out_specs=pl.BlockSpec((GB, 1, d_out), lambda i: (i, 0, 0)),
        out_shape=jax.ShapeDtypeStruct((B, 1, d_out), jnp.float32),
    )(adj, v_feature.astype(jnp.float32), W1, b1r, W2, b2r)
    return out
